# baseline (speedup 1.0000x reference)
import functools

import jax
import jax.numpy as jnp
from jax import lax
from jax.experimental import pallas as pl
from jax.experimental.pallas import tpu as pltpu
from jax.experimental.pallas import tpu_sc as plsc

NUM_TOKENS = 32768
HIDDEN = 768
N_EXPERTS = 64
TOP_K = 8
BLOCK_T = 2048
NW = 32
CHUNK = NUM_TOKENS // NW
LANES = 16


def _scores_body(x_ref, w_ref, s_ref):
    logits = jax.lax.dot_general(
        w_ref[...], x_ref[...], (((1,), (1,)), ((), ())),
        preferred_element_type=jnp.float32,
    )
    s_ref[...] = jax.nn.sigmoid(logits)


def _tc_scores(hidden_states, gate_weight):
    n_blocks = NUM_TOKENS // BLOCK_T
    return pl.pallas_call(
        _scores_body,
        grid=(n_blocks,),
        in_specs=[
            pl.BlockSpec((BLOCK_T, HIDDEN), lambda i: (i, 0)),
            pl.BlockSpec((N_EXPERTS, HIDDEN), lambda i: (0, 0)),
        ],
        out_specs=pl.BlockSpec((N_EXPERTS, BLOCK_T), lambda i: (0, i)),
        out_shape=jax.ShapeDtypeStruct((N_EXPERTS, NUM_TOKENS), jnp.float32),
    )(hidden_states, gate_weight)


def _sc_topk_body(s_hbm, w_hbm, i_hbm, sbuf, wbuf, ibuf, sem):
    wid = lax.axis_index("s") * 2 + lax.axis_index("c")
    base = wid * CHUNK
    copies = [
        pltpu.async_copy(s_hbm.at[e, pl.ds(base, CHUNK)], sbuf.at[e], sem)
        for e in range(N_EXPERTS)
    ]
    for c in copies:
        c.wait()

    def group(g, carry_unused):
        off = g * LANES
        neg = jnp.full((LANES,), -1.0, jnp.float32)
        vals = [neg] * TOP_K
        idxs = [jnp.full((LANES,), 0, jnp.int32)] * TOP_K
        for e in range(N_EXPERTS):
            v = sbuf[e, pl.ds(off, LANES)]
            i = jnp.full((LANES,), e, jnp.int32)
            for j in range(TOP_K):
                c = v > vals[j]
                nv = jnp.where(c, v, vals[j])
                ni = jnp.where(c, i, idxs[j])
                v = jnp.where(c, vals[j], v)
                i = jnp.where(c, idxs[j], i)
                vals[j] = nv
                idxs[j] = ni
        denom = vals[0]
        for j in range(1, TOP_K):
            denom = denom + vals[j]
        denom = denom + 1e-20
        for j in range(TOP_K):
            wbuf[j, pl.ds(off, LANES)] = vals[j] / denom
            ibuf[j, pl.ds(off, LANES)] = idxs[j]
        return carry_unused

    lax.fori_loop(0, CHUNK // LANES, group, 0)

    out_copies = [
        pltpu.async_copy(wbuf.at[j], w_hbm.at[j, pl.ds(base, CHUNK)], sem)
        for j in range(TOP_K)
    ] + [
        pltpu.async_copy(ibuf.at[j], i_hbm.at[j, pl.ds(base, CHUNK)], sem)
        for j in range(TOP_K)
    ]
    for c in out_copies:
        c.wait()


def _sc_topk(scores_t):
    fn = functools.partial(
        pl.kernel,
        mesh=plsc.VectorSubcoreMesh(core_axis_name="c", subcore_axis_name="s"),
        out_type=[
            jax.ShapeDtypeStruct((TOP_K, NUM_TOKENS), jnp.float32),
            jax.ShapeDtypeStruct((TOP_K, NUM_TOKENS), jnp.int32),
        ],
        scratch_types=[
            pltpu.VMEM((N_EXPERTS, CHUNK), jnp.float32),
            pltpu.VMEM((TOP_K, CHUNK), jnp.float32),
            pltpu.VMEM((TOP_K, CHUNK), jnp.int32),
            pltpu.SemaphoreType.DMA,
        ],
    )(_sc_topk_body)
    return fn(scores_t)


def kernel(hidden_states, gate_weight):
    scores_t = _tc_scores(hidden_states, gate_weight)
    wv_t, iv_t = _sc_topk(scores_t)
    return wv_t.T, iv_t.T


# speedup vs baseline: 2.2473x; 2.2473x over previous
import jax
import jax.numpy as jnp
from jax.experimental import pallas as pl

NUM_TOKENS = 32768
HIDDEN = 768
N_EXPERTS = 64
TOP_K = 8
BLOCK_T = 4096


def _router_body(x_ref, w_ref, wout_ref, iout_ref):
    x = x_ref[...]
    w = w_ref[...]
    logits = jax.lax.dot_general(
        w, x, (((1,), (1,)), ((), ())), preferred_element_type=jnp.float32
    )
    s = jax.nn.sigmoid(logits)
    rows = jax.lax.broadcasted_iota(jnp.int32, s.shape, 0)
    vals = []
    idxs = []
    for k in range(TOP_K):
        m = jnp.max(s, axis=0)
        idx = jnp.argmax(s, axis=0)
        vals.append(m)
        idxs.append(idx)
        if k + 1 < TOP_K:
            s = jnp.where(rows == idx[None, :], -1.0, s)
    wv = jnp.stack(vals, axis=0)
    iv = jnp.stack(idxs, axis=0)
    denom = jnp.sum(wv, axis=0, keepdims=True) + 1e-20
    wout_ref[...] = wv / denom
    iout_ref[...] = iv


def kernel(hidden_states, gate_weight):
    n_blocks = NUM_TOKENS // BLOCK_T
    wv_t, iv_t = pl.pallas_call(
        _router_body,
        grid=(n_blocks,),
        in_specs=[
            pl.BlockSpec((BLOCK_T, HIDDEN), lambda i: (i, 0)),
            pl.BlockSpec((N_EXPERTS, HIDDEN), lambda i: (0, 0)),
        ],
        out_specs=[
            pl.BlockSpec((TOP_K, BLOCK_T), lambda i: (0, i)),
            pl.BlockSpec((TOP_K, BLOCK_T), lambda i: (0, i)),
        ],
        out_shape=[
            jax.ShapeDtypeStruct((TOP_K, NUM_TOKENS), jnp.float32),
            jax.ShapeDtypeStruct((TOP_K, NUM_TOKENS), jnp.int32),
        ],
    )(hidden_states, gate_weight)
    return wv_t.T, iv_t.T
